# Initial kernel scaffold; baseline (speedup 1.0000x reference)
import jax
import jax.numpy as jnp
from jax.experimental import pallas as pl


def kernel(text, seq_len, emb_table):
    raise NotImplementedError("write your pallas kernel here")


# speedup vs baseline: 7.6194x; 7.6194x over previous
import functools

import jax
import jax.numpy as jnp
from jax import lax
from jax.experimental import pallas as pl
from jax.experimental.pallas import tpu as pltpu
from jax.experimental.pallas import tpu_sc as plsc


def _gather_kernel(n_rows, dim, n_workers, chunk):
    n_sub = chunk // 128
    rows_per_worker = n_rows // n_workers
    n_chunks = rows_per_worker // chunk
    nc = plsc.get_sparse_core_info().num_cores

    @functools.partial(
        pl.kernel,
        mesh=plsc.VectorSubcoreMesh(core_axis_name="c", subcore_axis_name="s"),
        out_type=jax.ShapeDtypeStruct((n_rows, dim), jnp.float32),
        scratch_types=[
            pltpu.VMEM((2, n_sub, 128), jnp.int32),
            pltpu.VMEM((2, chunk, dim), jnp.float32),
            pltpu.VMEM((16,), jnp.int32),
            pltpu.SemaphoreType.DMA,
            pltpu.SemaphoreType.DMA,
            pltpu.SemaphoreType.DMA,
            pltpu.SemaphoreType.DMA,
        ],
    )
    def k(idx_hbm, table_hbm, off_hbm, out_hbm, idx_v, rows_v, off_v, g0, g1, o0, o1):
        wid = lax.axis_index("s") * nc + lax.axis_index("c")
        base_row = wid * (rows_per_worker // 128)
        gsem = (g0, g1)
        osem = (o0, o1)
        pltpu.sync_copy(off_hbm, off_v)
        offv = off_v[...]

        def load_and_fire(g, buf):
            pltpu.sync_copy(
                idx_hbm.at[pl.ds(base_row + g * n_sub, n_sub)],
                idx_v.at[buf],
            )
            for j in range(n_sub):
                for v in range(128 // 16):
                    sl = pl.ds(v * 16, 16)
                    idx_v[buf, j, sl] = idx_v[buf, j, sl] + offv
            cps = []
            for j in range(n_sub):
                cps.append(
                    pltpu.async_copy(
                        table_hbm.at[idx_v.at[buf, j]],
                        rows_v.at[buf, pl.ds(j * 128, 128)],
                        gsem[buf],
                    )
                )
            return cps

        out_cps = [None, None]
        gat_cps = load_and_fire(0, 0)
        for g in range(n_chunks):
            buf = g % 2
            nxt = (g + 1) % 2
            if g + 1 < n_chunks:
                if out_cps[nxt] is not None:
                    out_cps[nxt].wait()
                    out_cps[nxt] = None
                next_gat = load_and_fire(g + 1, nxt)
            for cp in gat_cps:
                cp.wait()
            out_cps[buf] = pltpu.async_copy(
                rows_v.at[buf],
                out_hbm.at[pl.ds(wid * rows_per_worker + g * chunk, chunk)],
                osem[buf],
            )
            if g + 1 < n_chunks:
                gat_cps = next_gat
        for cp in out_cps:
            if cp is not None:
                cp.wait()

    return k


def kernel(text, seq_len, emb_table):
    batch, max_seq = text.shape
    _, dim = emb_table.shape
    n_rows = batch * max_seq
    n_workers = 32
    chunk = 256
    off = jnp.asarray(seq_len, jnp.int32) + jnp.int32(1 - max_seq)
    off_arr = jnp.broadcast_to(off, (16,))
    idx2d = text.reshape(n_rows // 128, 128)
    out = _gather_kernel(n_rows, dim, n_workers, chunk)(idx2d, emb_table, off_arr)
    return out.reshape(batch, max_seq, dim)


# speedup vs baseline: 7.6962x; 1.0101x over previous
import functools

import jax
import jax.numpy as jnp
from jax import lax
from jax.experimental import pallas as pl
from jax.experimental.pallas import tpu as pltpu
from jax.experimental.pallas import tpu_sc as plsc


def _gather_kernel(n_rows, dim, n_workers, chunk, nbuf):
    n_sub = chunk // 128
    rows_per_worker = n_rows // n_workers
    idx_rows = rows_per_worker // 128
    n_chunks = rows_per_worker // chunk
    nc = plsc.get_sparse_core_info().num_cores

    @functools.partial(
        pl.kernel,
        mesh=plsc.VectorSubcoreMesh(core_axis_name="c", subcore_axis_name="s"),
        out_type=jax.ShapeDtypeStruct((n_rows, dim), jnp.float32),
        scratch_types=[
            pltpu.VMEM((idx_rows, 128), jnp.int32),
            pltpu.VMEM((nbuf, chunk, dim), jnp.float32),
            pltpu.VMEM((16,), jnp.int32),
            pltpu.SemaphoreType.DMA,
        ]
        + [pltpu.SemaphoreType.DMA] * nbuf
        + [pltpu.SemaphoreType.DMA] * nbuf,
    )
    def k(idx_hbm, table_hbm, off_hbm, out_hbm, idx_v, rows_v, off_v, isem, *sems):
        gsem = sems[:nbuf]
        osem = sems[nbuf:]
        wid = lax.axis_index("s") * nc + lax.axis_index("c")
        pltpu.sync_copy(off_hbm, off_v)
        pltpu.async_copy(idx_hbm.at[wid], idx_v, isem).wait()
        offv = off_v[...]
        for j in range(idx_rows):
            for v in range(128 // 16):
                sl = pl.ds(v * 16, 16)
                idx_v[j, sl] = idx_v[j, sl] + offv

        def fire_gather(g):
            buf = g % nbuf
            return [
                pltpu.async_copy(
                    table_hbm.at[idx_v.at[g * n_sub + j]],
                    rows_v.at[buf, pl.ds(j * 128, 128)],
                    gsem[buf],
                )
                for j in range(n_sub)
            ]

        out_cps = [None] * nbuf
        gat_cps = [None] * nbuf
        for g in range(min(nbuf, n_chunks)):
            gat_cps[g % nbuf] = fire_gather(g)
        for g in range(n_chunks):
            buf = g % nbuf
            prev = g - 1
            nxt = prev + nbuf
            if prev >= 0 and nxt < n_chunks:
                out_cps[prev % nbuf].wait()
                out_cps[prev % nbuf] = None
                gat_cps[prev % nbuf] = fire_gather(nxt)
            for cp in gat_cps[buf]:
                cp.wait()
            out_cps[buf] = pltpu.async_copy(
                rows_v.at[buf],
                out_hbm.at[pl.ds(wid * rows_per_worker + g * chunk, chunk)],
                osem[buf],
            )
        for cp in out_cps:
            if cp is not None:
                cp.wait()

    return k


def kernel(text, seq_len, emb_table):
    batch, max_seq = text.shape
    _, dim = emb_table.shape
    n_rows = batch * max_seq
    n_workers = 32
    chunk = 256
    nbuf = 3
    off = jnp.asarray(seq_len, jnp.int32) + jnp.int32(1 - max_seq)
    off_arr = jnp.broadcast_to(off, (16,))
    idx3d = text.reshape(n_workers, n_rows // n_workers // 128, 128)
    out = _gather_kernel(n_rows, dim, n_workers, chunk, nbuf)(idx3d, emb_table, off_arr)
    return out.reshape(batch, max_seq, dim)
